# baseline (speedup 1.0000x reference)
import jax
import jax.numpy as jnp
from jax import lax
from jax.experimental import pallas as pl
from jax.experimental.pallas import tpu as pltpu
from jax.experimental.pallas import tpu_sc as plsc

_LR_MUL = 0.1
_PLACEHOLDER = 265
_BA = 256
_BS = 1232
_NC = 2
_NS = 16


def _mlp_body(s_ref, tok_ref, face_ref, w1_ref, b1_ref, w2_ref, b2_ref,
              cm_ref, cs_ref, text_ref, idx_ref):
    s = s_ref[0, 0]
    x = face_ref[...]
    nrm = jnp.sqrt(jnp.sum(x * x, axis=1, keepdims=True))
    x = x / jnp.maximum(nrm, 1e-12)
    h = jax.lax.dot_general(x, w1_ref[...], (((1,), (1,)), ((), ())),
                            preferred_element_type=jnp.float32)
    h = h * _LR_MUL + b1_ref[...] * _LR_MUL
    h = jnp.where(h > 0, h, h * 0.2)
    h = jax.lax.dot_general(h, w2_ref[...], (((1,), (1,)), ((), ())),
                            preferred_element_type=jnp.float32)
    h = h * _LR_MUL + b2_ref[...] * _LR_MUL
    h = jnp.where(h > 0, h, h * 0.2)
    text_ref[...] = (cm_ref[...] + h * cs_ref[...]) * s
    tok = tok_ref[...]
    ba, seq = tok.shape
    nb = pl.num_programs(0) * ba
    col = lax.broadcasted_iota(jnp.int32, tok.shape, 1)
    pos = jnp.min(jnp.where(tok == _PLACEHOLDER, col, seq), axis=1,
                  keepdims=True)
    b_glob = (lax.broadcasted_iota(jnp.int32, (ba, 2), 0)
              + pl.program_id(0) * ba)
    j = lax.broadcasted_iota(jnp.int32, (ba, 2), 1)
    idx_ref[...] = jnp.minimum((pos + j) * nb + b_glob, seq * nb - 1)


def _scale_body(s_ref, emb_ref, out_ref):
    out_ref[...] = emb_ref[...] * s_ref[0, 0]


def _sc_scatter_body(idx_hbm, vals_hbm, out_hbm, idx_v, vals_v, sem):
    wid = lax.axis_index("s") * _NC + lax.axis_index("c")
    n = idx_hbm.shape[0] // (_NC * _NS)
    base = wid * n
    pltpu.sync_copy(idx_hbm.at[pl.ds(base, n)], idx_v)
    pltpu.sync_copy(vals_hbm.at[pl.ds(base, n)], vals_v)
    pltpu.async_copy(vals_v, out_hbm.at[idx_v], sem).wait()


def kernel(tokenized_text, embedded_text, tokenizer_id, face_img_embeddings,
           W1, b1, W2, b2, celeb_mean, celeb_std):
    B, S, D = embedded_text.shape
    H = W1.shape[0]
    V = W1.shape[1]
    s = jnp.asarray(tokenizer_id, embedded_text.dtype).reshape(1, 1)
    b1r = b1.reshape(1, H)
    b2r = b2.reshape(1, H)
    cm = celeb_mean.reshape(1, H)
    cs = celeb_std.reshape(1, H)

    text, idx2 = pl.pallas_call(
        _mlp_body,
        grid=(B // _BA,),
        in_specs=[
            pl.BlockSpec((1, 1), lambda i: (0, 0), memory_space=pltpu.SMEM),
            pl.BlockSpec((_BA, S), lambda i: (i, 0)),
            pl.BlockSpec((_BA, V), lambda i: (i, 0)),
            pl.BlockSpec((H, V), lambda i: (0, 0)),
            pl.BlockSpec((1, H), lambda i: (0, 0)),
            pl.BlockSpec((H, H), lambda i: (0, 0)),
            pl.BlockSpec((1, H), lambda i: (0, 0)),
            pl.BlockSpec((1, H), lambda i: (0, 0)),
            pl.BlockSpec((1, H), lambda i: (0, 0)),
        ],
        out_specs=[
            pl.BlockSpec((_BA, H), lambda i: (i, 0)),
            pl.BlockSpec((_BA, 2), lambda i: (i, 0)),
        ],
        out_shape=[
            jax.ShapeDtypeStruct((B, H), embedded_text.dtype),
            jax.ShapeDtypeStruct((B, 2), jnp.int32),
        ],
    )(s, tokenized_text, face_img_embeddings, W1, b1r, W2, b2r, cm, cs)

    emb_flat = jnp.transpose(embedded_text, (1, 0, 2)).reshape(S * B, D)
    out_flat = pl.pallas_call(
        _scale_body,
        grid=(S * B // _BS,),
        in_specs=[
            pl.BlockSpec((1, 1), lambda i: (0, 0), memory_space=pltpu.SMEM),
            pl.BlockSpec((_BS, D), lambda i: (i, 0)),
        ],
        out_specs=pl.BlockSpec((_BS, D), lambda i: (i, 0)),
        out_shape=jax.ShapeDtypeStruct((S * B, D), embedded_text.dtype),
    )(s, emb_flat)

    mesh = plsc.VectorSubcoreMesh(core_axis_name="c", subcore_axis_name="s",
                                  num_cores=_NC, num_subcores=_NS)
    npw = (2 * B) // (_NC * _NS)
    scatter = pl.kernel(
        _sc_scatter_body,
        out_type=(),
        mesh=mesh,
        scratch_types=[
            pltpu.VMEM((npw,), jnp.int32),
            pltpu.VMEM((npw, D), jnp.float32),
            pltpu.SemaphoreType.DMA,
        ],
    )
    out_ref = jax.new_ref(out_flat)
    scatter(idx2.reshape(2 * B), text.reshape(2 * B, D), out_ref)
    out = jax.freeze(out_ref)
    return jnp.transpose(out.reshape(S, B, D), (1, 0, 2))


# speedup vs baseline: 1.0160x; 1.0160x over previous
import jax
import jax.numpy as jnp
from jax import lax
from jax.experimental import pallas as pl
from jax.experimental.pallas import tpu as pltpu
from jax.experimental.pallas import tpu_sc as plsc

_LR_MUL = 0.1
_PLACEHOLDER = 265
_BA = 256
_BS = 4928
_NC = 2
_NS = 16


def _mlp_body(s_ref, tok_ref, face_ref, w1_ref, b1_ref, w2_ref, b2_ref,
              cm_ref, cs_ref, text_ref, idx_ref):
    s = s_ref[0, 0]
    x = face_ref[...]
    nrm = jnp.sqrt(jnp.sum(x * x, axis=1, keepdims=True))
    x = x / jnp.maximum(nrm, 1e-12)
    h = jax.lax.dot_general(x, w1_ref[...], (((1,), (1,)), ((), ())),
                            preferred_element_type=jnp.float32)
    h = h * _LR_MUL + b1_ref[...] * _LR_MUL
    h = jnp.where(h > 0, h, h * 0.2)
    h = jax.lax.dot_general(h, w2_ref[...], (((1,), (1,)), ((), ())),
                            preferred_element_type=jnp.float32)
    h = h * _LR_MUL + b2_ref[...] * _LR_MUL
    h = jnp.where(h > 0, h, h * 0.2)
    text_ref[...] = (cm_ref[...] + h * cs_ref[...]) * s
    tok = tok_ref[...]
    ba, seq = tok.shape
    nb = pl.num_programs(0) * ba
    col = lax.broadcasted_iota(jnp.int32, tok.shape, 1)
    pos = jnp.min(jnp.where(tok == _PLACEHOLDER, col, seq), axis=1,
                  keepdims=True)
    b_glob = (lax.broadcasted_iota(jnp.int32, (ba, 2), 0)
              + pl.program_id(0) * ba)
    j = lax.broadcasted_iota(jnp.int32, (ba, 2), 1)
    idx_ref[...] = jnp.minimum((pos + j) * nb + b_glob, seq * nb - 1)


def _scale_body(s_ref, emb_ref, out_ref):
    out_ref[...] = emb_ref[...] * s_ref[0, 0]


def _sc_scatter_body(idx_hbm, vals_hbm, out_hbm, idx_v, vals_v, sem):
    wid = lax.axis_index("s") * _NC + lax.axis_index("c")
    n = idx_hbm.shape[0] // (_NC * _NS)
    base = wid * n
    pltpu.sync_copy(idx_hbm.at[pl.ds(base, n)], idx_v)
    pltpu.sync_copy(vals_hbm.at[pl.ds(base, n)], vals_v)
    pltpu.async_copy(vals_v, out_hbm.at[idx_v], sem).wait()


def kernel(tokenized_text, embedded_text, tokenizer_id, face_img_embeddings,
           W1, b1, W2, b2, celeb_mean, celeb_std):
    B, S, D = embedded_text.shape
    H = W1.shape[0]
    V = W1.shape[1]
    s = jnp.asarray(tokenizer_id, embedded_text.dtype).reshape(1, 1)
    b1r = b1.reshape(1, H)
    b2r = b2.reshape(1, H)
    cm = celeb_mean.reshape(1, H)
    cs = celeb_std.reshape(1, H)

    text, idx2 = pl.pallas_call(
        _mlp_body,
        grid=(B // _BA,),
        in_specs=[
            pl.BlockSpec((1, 1), lambda i: (0, 0), memory_space=pltpu.SMEM),
            pl.BlockSpec((_BA, S), lambda i: (i, 0)),
            pl.BlockSpec((_BA, V), lambda i: (i, 0)),
            pl.BlockSpec((H, V), lambda i: (0, 0)),
            pl.BlockSpec((1, H), lambda i: (0, 0)),
            pl.BlockSpec((H, H), lambda i: (0, 0)),
            pl.BlockSpec((1, H), lambda i: (0, 0)),
            pl.BlockSpec((1, H), lambda i: (0, 0)),
            pl.BlockSpec((1, H), lambda i: (0, 0)),
        ],
        out_specs=[
            pl.BlockSpec((_BA, H), lambda i: (i, 0)),
            pl.BlockSpec((_BA, 2), lambda i: (i, 0)),
        ],
        out_shape=[
            jax.ShapeDtypeStruct((B, H), embedded_text.dtype),
            jax.ShapeDtypeStruct((B, 2), jnp.int32),
        ],
    )(s, tokenized_text, face_img_embeddings, W1, b1r, W2, b2r, cm, cs)

    emb_flat = jnp.transpose(embedded_text, (1, 0, 2)).reshape(S * B, D)
    out_flat = pl.pallas_call(
        _scale_body,
        grid=(S * B // _BS,),
        in_specs=[
            pl.BlockSpec((1, 1), lambda i: (0, 0), memory_space=pltpu.SMEM),
            pl.BlockSpec((_BS, D), lambda i: (i, 0)),
        ],
        out_specs=pl.BlockSpec((_BS, D), lambda i: (i, 0)),
        out_shape=jax.ShapeDtypeStruct((S * B, D), embedded_text.dtype),
    )(s, emb_flat)

    mesh = plsc.VectorSubcoreMesh(core_axis_name="c", subcore_axis_name="s",
                                  num_cores=_NC, num_subcores=_NS)
    npw = (2 * B) // (_NC * _NS)
    scatter = pl.kernel(
        _sc_scatter_body,
        out_type=(),
        mesh=mesh,
        scratch_types=[
            pltpu.VMEM((npw,), jnp.int32),
            pltpu.VMEM((npw, D), jnp.float32),
            pltpu.SemaphoreType.DMA,
        ],
    )
    out_ref = jax.new_ref(out_flat)
    scatter(idx2.reshape(2 * B), text.reshape(2 * B, D), out_ref)
    out = jax.freeze(out_ref)
    return jnp.transpose(out.reshape(S, B, D), (1, 0, 2))


# speedup vs baseline: 1.0484x; 1.0319x over previous
import jax
import jax.numpy as jnp
from jax import lax
from jax.experimental import pallas as pl
from jax.experimental.pallas import tpu as pltpu
from jax.experimental.pallas import tpu_sc as plsc

_LR_MUL = 0.1
_PLACEHOLDER = 265
_BA = 256
_BS = 2464
_NC = 2
_NS = 16


def _sc_scatter_body(idx_hbm, vals_hbm, out_hbm, idx_v, vals_v, sem):
    wid = lax.axis_index("s") * _NC + lax.axis_index("c")
    n = idx_hbm.shape[0] // (_NC * _NS)
    base = wid * n
    pltpu.sync_copy(idx_hbm.at[pl.ds(base, n)], idx_v)
    pltpu.sync_copy(vals_hbm.at[pl.ds(base, n)], vals_v)
    pltpu.async_copy(vals_v, out_hbm.at[idx_v], sem).wait()


def kernel(tokenized_text, embedded_text, tokenizer_id, face_img_embeddings,
           W1, b1, W2, b2, celeb_mean, celeb_std):
    B, S, D = embedded_text.shape
    H = W1.shape[0]
    V = W1.shape[1]
    nmlp = B // _BA
    s = jnp.asarray(tokenizer_id, embedded_text.dtype).reshape(1, 1)
    b1r = b1.reshape(1, H)
    b2r = b2.reshape(1, H)
    cm = celeb_mean.reshape(1, H)
    cs = celeb_std.reshape(1, H)

    def _fused_body(s_ref, tok_ref, face_ref, w1_ref, b1_ref, w2_ref, b2_ref,
                    cm_ref, cs_ref, emb_ref, out_ref, text_ref, idx_ref):
        sc = s_ref[0, 0]
        out_ref[...] = emb_ref[...] * sc
        i = pl.program_id(0)

        @pl.when(i < nmlp)
        def _mlp():
            x = face_ref[...]
            nrm = jnp.sqrt(jnp.sum(x * x, axis=1, keepdims=True))
            x = x / jnp.maximum(nrm, 1e-12)
            h = jax.lax.dot_general(x, w1_ref[...], (((1,), (1,)), ((), ())),
                                    preferred_element_type=jnp.float32)
            h = h * _LR_MUL + b1_ref[...] * _LR_MUL
            h = jnp.where(h > 0, h, h * 0.2)
            h = jax.lax.dot_general(h, w2_ref[...], (((1,), (1,)), ((), ())),
                                    preferred_element_type=jnp.float32)
            h = h * _LR_MUL + b2_ref[...] * _LR_MUL
            h = jnp.where(h > 0, h, h * 0.2)
            text_ref[...] = (cm_ref[...] + h * cs_ref[...]) * sc
            tok = tok_ref[...]
            col = lax.broadcasted_iota(jnp.int32, tok.shape, 1)
            pos = jnp.min(jnp.where(tok == _PLACEHOLDER, col, S), axis=1,
                          keepdims=True)
            b_glob = lax.broadcasted_iota(jnp.int32, (_BA, 2), 0) + i * _BA
            j = lax.broadcasted_iota(jnp.int32, (_BA, 2), 1)
            idx_ref[...] = jnp.minimum((pos + j) * B + b_glob, S * B - 1)

    emb_flat = jnp.transpose(embedded_text, (1, 0, 2)).reshape(S * B, D)
    mcap = nmlp - 1
    out_flat, text, idx2 = pl.pallas_call(
        _fused_body,
        grid=(S * B // _BS,),
        in_specs=[
            pl.BlockSpec((1, 1), lambda i: (0, 0), memory_space=pltpu.SMEM),
            pl.BlockSpec((_BA, S), lambda i: (jnp.minimum(i, mcap), 0)),
            pl.BlockSpec((_BA, V), lambda i: (jnp.minimum(i, mcap), 0)),
            pl.BlockSpec((H, V), lambda i: (0, 0)),
            pl.BlockSpec((1, H), lambda i: (0, 0)),
            pl.BlockSpec((H, H), lambda i: (0, 0)),
            pl.BlockSpec((1, H), lambda i: (0, 0)),
            pl.BlockSpec((1, H), lambda i: (0, 0)),
            pl.BlockSpec((1, H), lambda i: (0, 0)),
            pl.BlockSpec((_BS, D), lambda i: (i, 0)),
        ],
        out_specs=[
            pl.BlockSpec((_BS, D), lambda i: (i, 0)),
            pl.BlockSpec((_BA, H), lambda i: (jnp.minimum(i, mcap), 0)),
            pl.BlockSpec((_BA, 2), lambda i: (jnp.minimum(i, mcap), 0)),
        ],
        out_shape=[
            jax.ShapeDtypeStruct((S * B, D), embedded_text.dtype),
            jax.ShapeDtypeStruct((B, H), embedded_text.dtype),
            jax.ShapeDtypeStruct((B, 2), jnp.int32),
        ],
    )(s, tokenized_text, face_img_embeddings, W1, b1r, W2, b2r, cm, cs,
      emb_flat)

    mesh = plsc.VectorSubcoreMesh(core_axis_name="c", subcore_axis_name="s",
                                  num_cores=_NC, num_subcores=_NS)
    npw = (2 * B) // (_NC * _NS)
    scatter = pl.kernel(
        _sc_scatter_body,
        out_type=(),
        mesh=mesh,
        scratch_types=[
            pltpu.VMEM((npw,), jnp.int32),
            pltpu.VMEM((npw, D), jnp.float32),
            pltpu.SemaphoreType.DMA,
        ],
    )
    out_ref = jax.new_ref(out_flat)
    scatter(idx2.reshape(2 * B), text.reshape(2 * B, D), out_ref)
    out = jax.freeze(out_ref)
    return jnp.transpose(out.reshape(S, B, D), (1, 0, 2))


# speedup vs baseline: 1.0526x; 1.0040x over previous
import jax
import jax.numpy as jnp
from jax import lax
from jax.experimental import pallas as pl
from jax.experimental.pallas import tpu as pltpu
from jax.experimental.pallas import tpu_sc as plsc

_LR_MUL = 0.1
_PLACEHOLDER = 265
_BA = 256
_BS = 2816
_NC = 2
_NS = 16


def _sc_scatter_body(idx_hbm, vals_hbm, out_hbm, idx_v, vals_v, sem):
    wid = lax.axis_index("s") * _NC + lax.axis_index("c")
    n = idx_hbm.shape[0] // (_NC * _NS)
    base = wid * n
    pltpu.sync_copy(idx_hbm.at[pl.ds(base, n)], idx_v)
    pltpu.sync_copy(vals_hbm.at[pl.ds(base, n)], vals_v)
    pltpu.async_copy(vals_v, out_hbm.at[idx_v], sem).wait()


def kernel(tokenized_text, embedded_text, tokenizer_id, face_img_embeddings,
           W1, b1, W2, b2, celeb_mean, celeb_std):
    B, S, D = embedded_text.shape
    H = W1.shape[0]
    V = W1.shape[1]
    nmlp = B // _BA
    s = jnp.asarray(tokenizer_id, embedded_text.dtype).reshape(1, 1)
    b1r = b1.reshape(1, H)
    b2r = b2.reshape(1, H)
    cm = celeb_mean.reshape(1, H)
    cs = celeb_std.reshape(1, H)

    def _fused_body(s_ref, tok_ref, face_ref, w1_ref, b1_ref, w2_ref, b2_ref,
                    cm_ref, cs_ref, emb_ref, out_ref, text_ref, idx_ref):
        sc = s_ref[0, 0]
        out_ref[...] = emb_ref[...] * sc
        i = pl.program_id(0)

        @pl.when(i < nmlp)
        def _mlp():
            x = face_ref[...]
            nrm = jnp.sqrt(jnp.sum(x * x, axis=1, keepdims=True))
            x = x / jnp.maximum(nrm, 1e-12)
            h = jax.lax.dot_general(x, w1_ref[...], (((1,), (1,)), ((), ())),
                                    preferred_element_type=jnp.float32)
            h = h * _LR_MUL + b1_ref[...] * _LR_MUL
            h = jnp.where(h > 0, h, h * 0.2)
            h = jax.lax.dot_general(h, w2_ref[...], (((1,), (1,)), ((), ())),
                                    preferred_element_type=jnp.float32)
            h = h * _LR_MUL + b2_ref[...] * _LR_MUL
            h = jnp.where(h > 0, h, h * 0.2)
            text_ref[...] = (cm_ref[...] + h * cs_ref[...]) * sc
            tok = tok_ref[...]
            col = lax.broadcasted_iota(jnp.int32, tok.shape, 1)
            pos = jnp.min(jnp.where(tok == _PLACEHOLDER, col, S), axis=1,
                          keepdims=True)
            b_glob = lax.broadcasted_iota(jnp.int32, (_BA, 2), 0) + i * _BA
            j = lax.broadcasted_iota(jnp.int32, (_BA, 2), 1)
            idx_ref[...] = jnp.minimum((pos + j) * B + b_glob, S * B - 1)

    emb_flat = jnp.transpose(embedded_text, (1, 0, 2)).reshape(S * B, D)
    mcap = nmlp - 1
    out_flat, text, idx2 = pl.pallas_call(
        _fused_body,
        grid=(S * B // _BS,),
        in_specs=[
            pl.BlockSpec((1, 1), lambda i: (0, 0), memory_space=pltpu.SMEM),
            pl.BlockSpec((_BA, S), lambda i: (jnp.minimum(i, mcap), 0)),
            pl.BlockSpec((_BA, V), lambda i: (jnp.minimum(i, mcap), 0)),
            pl.BlockSpec((H, V), lambda i: (0, 0)),
            pl.BlockSpec((1, H), lambda i: (0, 0)),
            pl.BlockSpec((H, H), lambda i: (0, 0)),
            pl.BlockSpec((1, H), lambda i: (0, 0)),
            pl.BlockSpec((1, H), lambda i: (0, 0)),
            pl.BlockSpec((1, H), lambda i: (0, 0)),
            pl.BlockSpec((_BS, D), lambda i: (i, 0)),
        ],
        out_specs=[
            pl.BlockSpec((_BS, D), lambda i: (i, 0)),
            pl.BlockSpec((_BA, H), lambda i: (jnp.minimum(i, mcap), 0)),
            pl.BlockSpec((_BA, 2), lambda i: (jnp.minimum(i, mcap), 0)),
        ],
        out_shape=[
            jax.ShapeDtypeStruct((S * B, D), embedded_text.dtype),
            jax.ShapeDtypeStruct((B, H), embedded_text.dtype),
            jax.ShapeDtypeStruct((B, 2), jnp.int32),
        ],
    )(s, tokenized_text, face_img_embeddings, W1, b1r, W2, b2r, cm, cs,
      emb_flat)

    mesh = plsc.VectorSubcoreMesh(core_axis_name="c", subcore_axis_name="s",
                                  num_cores=_NC, num_subcores=_NS)
    npw = (2 * B) // (_NC * _NS)
    scatter = pl.kernel(
        _sc_scatter_body,
        out_type=(),
        mesh=mesh,
        scratch_types=[
            pltpu.VMEM((npw,), jnp.int32),
            pltpu.VMEM((npw, D), jnp.float32),
            pltpu.SemaphoreType.DMA,
        ],
    )
    out_ref = jax.new_ref(out_flat)
    scatter(idx2.reshape(2 * B), text.reshape(2 * B, D), out_ref)
    out = jax.freeze(out_ref)
    return jnp.transpose(out.reshape(S, B, D), (1, 0, 2))
